# Initial kernel scaffold; baseline (speedup 1.0000x reference)
import jax
import jax.numpy as jnp
from jax.experimental import pallas as pl


def kernel(x, scores, fcw, fcb):
    raise NotImplementedError("write your pallas kernel here")


# speedup vs baseline: 32.7780x; 32.7780x over previous
import functools

import jax
import jax.numpy as jnp
from jax import lax
from jax.experimental import pallas as pl
from jax.experimental.pallas import tpu as pltpu
from jax.experimental.pallas import tpu_sc as plsc

_N = 4194304
_J = _N // 2
_NB = 4096
_NC = 2
_NS = 16
_NW = _NC * _NS
_PER_W = _N // _NW
_CH = 16384
_NCH = _PER_W // _CH
_LANES = 16

_OUT_F = 2048
_IN_F = 2048
_BT = 256


@functools.lru_cache(maxsize=None)
def _make_hist(level):
    mesh = plsc.VectorSubcoreMesh(core_axis_name="c", subcore_axis_name="s")
    scratch = [
        pltpu.VMEM((_CH,), jnp.float32),
        pltpu.VMEM((_LANES, _NB), jnp.int32),
        pltpu.VMEM((1, _NB), jnp.int32),
    ]
    if level > 1:
        scratch.append(pltpu.VMEM((_LANES,), jnp.int32))

    @functools.partial(
        pl.kernel,
        out_type=jax.ShapeDtypeStruct((_NW, _NB), jnp.int32),
        mesh=mesh,
        scratch_types=scratch,
        compiler_params=pltpu.CompilerParams(needs_layout_passes=False),
    )
    def hist_kernel(scores_hbm, *rest):
        if level > 1:
            prefix_hbm, out_hbm, buf, h16, hsum, pref_v = rest
            pltpu.sync_copy(prefix_hbm, pref_v)
            pref = pref_v[...]
        else:
            out_hbm, buf, h16, hsum = rest
            pref = None

        wid = lax.axis_index("s") * _NC + lax.axis_index("c")
        base = wid * _PER_W
        lanes = lax.iota(jnp.int32, _LANES)
        ones = jnp.ones((_LANES,), jnp.int32)
        zeros = jnp.zeros((_LANES,), jnp.int32)

        def zbody(k, _):
            for l in range(_LANES):
                h16[l, pl.ds(k * 16, 16)] = zeros
            return 0

        lax.fori_loop(0, _NB // 16, zbody, 0)

        def chunk_body(c, _):
            pltpu.sync_copy(scores_hbm.at[pl.ds(base + c * _CH, _CH)], buf)

            def vbody(k, _):
                v = buf[pl.ds(k * 16, 16)]
                bits = lax.bitcast_convert_type(v, jnp.int32) & jnp.int32(0x7FFFFFFF)
                if level == 1:
                    bins = bits >> 20
                    plsc.addupdate_scatter(h16, [lanes, bins], ones)
                elif level == 2:
                    valid = (bits >> 20) == pref
                    bins = (bits >> 8) & jnp.int32(0xFFF)
                    plsc.addupdate_scatter(h16, [lanes, bins], ones, mask=valid)
                else:
                    valid = (bits >> 8) == pref
                    bins = bits & jnp.int32(0xFF)
                    plsc.addupdate_scatter(h16, [lanes, bins], ones, mask=valid)
                return 0

            lax.fori_loop(0, _CH // 16, vbody, 0)
            return 0

        lax.fori_loop(0, _NCH, chunk_body, 0)

        def rbody(k, _):
            sl = pl.ds(k * 16, 16)
            acc = h16[0, sl]
            for l in range(1, _LANES):
                acc = acc + h16[l, sl]
            hsum[0, sl] = acc
            return 0

        lax.fori_loop(0, _NB // 16, rbody, 0)
        pltpu.sync_copy(hsum, out_hbm.at[pl.ds(wid, 1)])

    return hist_kernel


def _pick_kernel(hist_ref, rank_ref, out_ref):
    h = jnp.sum(hist_ref[...].astype(jnp.float32), axis=0)
    r = rank_ref[0, 0].astype(jnp.float32)

    rowi = lax.broadcasted_iota(jnp.int32, (32, 32), 0)
    coli = lax.broadcasted_iota(jnp.int32, (32, 32), 1)
    l_strict = (coli < rowi).astype(jnp.float32)
    m = jnp.dot(l_strict, h, preferred_element_type=jnp.float32,
                precision=lax.Precision.HIGHEST)
    prior = jnp.sum(m, axis=1, keepdims=True)

    ci = lax.broadcasted_iota(jnp.int32, (128, 128), 0)
    cj = lax.broadcasted_iota(jnp.int32, (128, 128), 1)
    u_incl = (ci <= cj).astype(jnp.float32)
    rowcum = jnp.dot(h, u_incl, preferred_element_type=jnp.float32,
                     precision=lax.Precision.HIGHEST)

    cum_incl = prior + rowcum
    cum_excl = cum_incl - h
    cond = (cum_excl <= r) & (r < cum_incl)

    binmat = (lax.broadcasted_iota(jnp.int32, (32, 128), 0) * 128
              + lax.broadcasted_iota(jnp.int32, (32, 128), 1)).astype(jnp.float32)
    sel = jnp.sum(jnp.where(cond, binmat, 0.0))
    newr = r - jnp.sum(jnp.where(cond, cum_excl, 0.0))

    lane = lax.broadcasted_iota(jnp.int32, (1, 128), 1)
    out_ref[...] = jnp.where(
        lane == 0, sel.astype(jnp.int32),
        jnp.where(lane == 1, newr.astype(jnp.int32), 0))


_pick = pl.pallas_call(
    _pick_kernel,
    out_shape=jax.ShapeDtypeStruct((1, 128), jnp.int32),
    in_specs=[
        pl.BlockSpec(memory_space=pltpu.VMEM),
        pl.BlockSpec(memory_space=pltpu.SMEM),
    ],
    out_specs=pl.BlockSpec(memory_space=pltpu.VMEM),
)


def _mm_kernel(t_ref, x_ref, scores_ref, fcw_ref, fcb_ref, o_ref, w_ref):
    @pl.when(pl.program_id(0) == 0)
    def _():
        bits = lax.bitcast_convert_type(scores_ref[...], jnp.int32)
        bits = bits & jnp.int32(0x7FFFFFFF)
        t = t_ref[0, 0]
        w_ref[...] = jnp.where(bits >= t, fcw_ref[...], 0.0)

    y = lax.dot_general(x_ref[...], w_ref[...],
                        (((1,), (1,)), ((), ())),
                        preferred_element_type=jnp.float32)
    o_ref[...] = y + fcb_ref[...]


def _masked_matmul(t11, x, scores, fcw, fcb2):
    n_tok = x.shape[0]
    grid = (n_tok // _BT,)
    return pl.pallas_call(
        _mm_kernel,
        grid=grid,
        out_shape=jax.ShapeDtypeStruct((n_tok, _OUT_F), jnp.float32),
        in_specs=[
            pl.BlockSpec((1, 1), lambda i: (0, 0), memory_space=pltpu.SMEM),
            pl.BlockSpec((_BT, _IN_F), lambda i: (i, 0)),
            pl.BlockSpec((_OUT_F, _IN_F), lambda i: (0, 0)),
            pl.BlockSpec((_OUT_F, _IN_F), lambda i: (0, 0)),
            pl.BlockSpec((1, _OUT_F), lambda i: (0, 0)),
        ],
        out_specs=pl.BlockSpec((_BT, _OUT_F), lambda i: (i, 0)),
        scratch_shapes=[pltpu.VMEM((_OUT_F, _IN_F), jnp.float32)],
    )(t11, x, scores, fcw, fcb2)


def kernel(x, scores, fcw, fcb):
    s_flat = scores.reshape(-1)
    r0 = jnp.full((1, 1), _J, jnp.int32)

    h1 = _make_hist(1)(s_flat)
    p1 = _pick(h1.reshape(_NW, 32, 128), r0)
    b1 = p1[0, 0]

    h2 = _make_hist(2)(s_flat, jnp.broadcast_to(b1, (_LANES,)))
    p2 = _pick(h2.reshape(_NW, 32, 128), p1[0:1, 1:2])
    b2 = p2[0, 0]

    h3 = _make_hist(3)(s_flat, jnp.broadcast_to((b1 << 12) | b2, (_LANES,)))
    p3 = _pick(h3.reshape(_NW, 32, 128), p2[0:1, 1:2])
    b3 = p3[0, 0]

    t = (b1 << 20) | (b2 << 8) | b3
    return _masked_matmul(t.reshape(1, 1), x, scores, fcw,
                          fcb.reshape(1, _OUT_F))


# speedup vs baseline: 35.0194x; 1.0684x over previous
import functools

import jax
import jax.numpy as jnp
from jax import lax
from jax.experimental import pallas as pl
from jax.experimental.pallas import tpu as pltpu
from jax.experimental.pallas import tpu_sc as plsc

_N = 4194304
_J = _N // 2
_NB = 4096
_NC = 2
_NS = 16
_NW = _NC * _NS
_PER_W = _N // _NW
_CH = 16384
_NCH = _PER_W // _CH
_LANES = 16
_UNROLL = 8

_OUT_F = 2048
_IN_F = 2048
_BT = 512


@functools.lru_cache(maxsize=None)
def _make_hist(level):
    mesh = plsc.VectorSubcoreMesh(core_axis_name="c", subcore_axis_name="s")
    scratch = [
        pltpu.VMEM((_CH,), jnp.float32),
        pltpu.VMEM((_LANES, _NB), jnp.int32),
        pltpu.VMEM((32, 128), jnp.int32),
    ]
    if level > 1:
        scratch.append(pltpu.VMEM((_LANES,), jnp.int32))

    @functools.partial(
        pl.kernel,
        out_type=jax.ShapeDtypeStruct((_NW, 32, 128), jnp.int32),
        mesh=mesh,
        scratch_types=scratch,
        compiler_params=pltpu.CompilerParams(needs_layout_passes=False),
    )
    def hist_kernel(scores_hbm, *rest):
        if level > 1:
            prefix_hbm, out_hbm, buf, h16, hsum, pref_v = rest
            pltpu.sync_copy(prefix_hbm, pref_v)
            pref = pref_v[...]
        else:
            out_hbm, buf, h16, hsum = rest
            pref = None

        wid = lax.axis_index("s") * _NC + lax.axis_index("c")
        base = wid * _PER_W
        lanes = lax.iota(jnp.int32, _LANES)
        ones = jnp.ones((_LANES,), jnp.int32)
        zeros = jnp.zeros((_LANES,), jnp.int32)

        def zbody(k, _):
            for l in range(_LANES):
                h16[l, pl.ds(k * 16, 16)] = zeros
            return 0

        lax.fori_loop(0, _NB // 16, zbody, 0)

        def one_vreg(off):
            v = buf[off]
            bits = lax.bitcast_convert_type(v, jnp.int32) & jnp.int32(0x7FFFFFFF)
            if level == 1:
                bins = bits >> 20
                plsc.addupdate_scatter(h16, [lanes, bins], ones)
            elif level == 2:
                valid = (bits >> 20) == pref
                bins = (bits >> 8) & jnp.int32(0xFFF)
                plsc.addupdate_scatter(h16, [lanes, bins], ones, mask=valid)
            else:
                valid = (bits >> 8) == pref
                bins = bits & jnp.int32(0xFF)
                plsc.addupdate_scatter(h16, [lanes, bins], ones, mask=valid)

        def chunk_body(c, _):
            pltpu.sync_copy(scores_hbm.at[pl.ds(base + c * _CH, _CH)], buf)

            def vbody(k, _):
                for u in range(_UNROLL):
                    one_vreg(pl.ds(k * (16 * _UNROLL) + u * 16, 16))
                return 0

            lax.fori_loop(0, _CH // (16 * _UNROLL), vbody, 0)
            return 0

        lax.fori_loop(0, _NCH, chunk_body, 0)

        def rbody(k, _):
            acc = h16[0, pl.ds(k * 16, 16)]
            for l in range(1, _LANES):
                acc = acc + h16[l, pl.ds(k * 16, 16)]
            hsum[k >> 3, pl.ds((k & 7) * 16, 16)] = acc
            return 0

        lax.fori_loop(0, _NB // 16, rbody, 0)
        pltpu.sync_copy(hsum, out_hbm.at[wid])

    return hist_kernel


def _pick_kernel(hist_ref, rank_ref, out_ref):
    h = jnp.sum(hist_ref[...].astype(jnp.float32), axis=0)
    r = rank_ref[0, 0].astype(jnp.float32)

    rowi = lax.broadcasted_iota(jnp.int32, (32, 32), 0)
    coli = lax.broadcasted_iota(jnp.int32, (32, 32), 1)
    l_strict = (coli < rowi).astype(jnp.float32)
    m = jnp.dot(l_strict, h, preferred_element_type=jnp.float32,
                precision=lax.Precision.HIGHEST)
    prior = jnp.sum(m, axis=1, keepdims=True)

    ci = lax.broadcasted_iota(jnp.int32, (128, 128), 0)
    cj = lax.broadcasted_iota(jnp.int32, (128, 128), 1)
    u_incl = (ci <= cj).astype(jnp.float32)
    rowcum = jnp.dot(h, u_incl, preferred_element_type=jnp.float32,
                     precision=lax.Precision.HIGHEST)

    cum_incl = prior + rowcum
    cum_excl = cum_incl - h
    cond = (cum_excl <= r) & (r < cum_incl)

    binmat = (lax.broadcasted_iota(jnp.int32, (32, 128), 0) * 128
              + lax.broadcasted_iota(jnp.int32, (32, 128), 1)).astype(jnp.float32)
    sel = jnp.sum(jnp.where(cond, binmat, 0.0))
    newr = r - jnp.sum(jnp.where(cond, cum_excl, 0.0))

    lane = lax.broadcasted_iota(jnp.int32, (1, 128), 1)
    out_ref[...] = jnp.where(
        lane == 0, sel.astype(jnp.int32),
        jnp.where(lane == 1, newr.astype(jnp.int32), 0))


_pick = pl.pallas_call(
    _pick_kernel,
    out_shape=jax.ShapeDtypeStruct((1, 128), jnp.int32),
    in_specs=[
        pl.BlockSpec(memory_space=pltpu.VMEM),
        pl.BlockSpec(memory_space=pltpu.SMEM),
    ],
    out_specs=pl.BlockSpec(memory_space=pltpu.VMEM),
)


def _mm_kernel(t_ref, x_ref, scores_ref, fcw_ref, fcb_ref, o_ref, w_ref):
    @pl.when(pl.program_id(0) == 0)
    def _():
        bits = lax.bitcast_convert_type(scores_ref[...], jnp.int32)
        bits = bits & jnp.int32(0x7FFFFFFF)
        t = t_ref[0, 0]
        w_ref[...] = jnp.where(bits >= t, fcw_ref[...], 0.0).astype(jnp.bfloat16)

    y = lax.dot_general(x_ref[...].astype(jnp.bfloat16), w_ref[...],
                        (((1,), (1,)), ((), ())),
                        preferred_element_type=jnp.float32)
    o_ref[...] = y + fcb_ref[...]


def _masked_matmul(t11, x, scores, fcw, fcb2):
    n_tok = x.shape[0]
    grid = (n_tok // _BT,)
    return pl.pallas_call(
        _mm_kernel,
        grid=grid,
        out_shape=jax.ShapeDtypeStruct((n_tok, _OUT_F), jnp.float32),
        in_specs=[
            pl.BlockSpec((1, 1), lambda i: (0, 0), memory_space=pltpu.SMEM),
            pl.BlockSpec((_BT, _IN_F), lambda i: (i, 0)),
            pl.BlockSpec((_OUT_F, _IN_F), lambda i: (0, 0)),
            pl.BlockSpec((_OUT_F, _IN_F), lambda i: (0, 0)),
            pl.BlockSpec((1, _OUT_F), lambda i: (0, 0)),
        ],
        out_specs=pl.BlockSpec((_BT, _OUT_F), lambda i: (i, 0)),
        scratch_shapes=[pltpu.VMEM((_OUT_F, _IN_F), jnp.bfloat16)],
    )(t11, x, scores, fcw, fcb2)


def kernel(x, scores, fcw, fcb):
    s_flat = scores.reshape(-1)
    r0 = jnp.full((1, 1), _J, jnp.int32)

    h1 = _make_hist(1)(s_flat)
    p1 = _pick(h1, r0)
    b1 = p1[0, 0]

    h2 = _make_hist(2)(s_flat, jnp.broadcast_to(b1, (_LANES,)))
    p2 = _pick(h2, p1[0:1, 1:2])
    b2 = p2[0, 0]

    h3 = _make_hist(3)(s_flat, jnp.broadcast_to((b1 << 12) | b2, (_LANES,)))
    p3 = _pick(h3, p2[0:1, 1:2])
    b3 = p3[0, 0]

    t = (b1 << 20) | (b2 << 8) | b3
    return _masked_matmul(t.reshape(1, 1), x, scores, fcw,
                          fcb.reshape(1, _OUT_F))
